# baseline (speedup 1.0000x reference)
import functools

import jax
import jax.numpy as jnp
from jax import lax
from jax.experimental import pallas as pl
from jax.experimental.pallas import tpu as pltpu
from jax.experimental.pallas import tpu_sc as plsc

_LB = 128
_DP = 128


@functools.lru_cache(maxsize=None)
def _make_gather(B, H, V, D):
    n_slab = B * H // _LB
    info = plsc.get_sparse_core_info()
    NC, NS = info.num_cores, info.num_subcores
    NW = NC * NS
    per_w = n_slab // NW
    assert per_w * NW == n_slab and per_w % 2 == 0
    tb_per_h = B // _LB
    mesh = plsc.VectorSubcoreMesh(core_axis_name="c", subcore_axis_name="s")

    @functools.partial(
        pl.kernel,
        mesh=mesh,
        compiler_params=pltpu.CompilerParams(needs_layout_passes=False),
        out_type=jax.ShapeDtypeStruct((H, D, B), jnp.float32),
        scratch_types=[
            pltpu.VMEM((2, 1, _LB), jnp.int32),
            pltpu.VMEM((2, _LB, _DP), jnp.float32),
            pltpu.VMEM((2, D, _LB), jnp.float32),
            pltpu.SemaphoreType.DMA,
            pltpu.SemaphoreType.DMA,
            pltpu.SemaphoreType.DMA,
            pltpu.SemaphoreType.DMA,
            pltpu.SemaphoreType.DMA,
            pltpu.SemaphoreType.DMA,
        ],
    )
    def k(idx_hbm, table_hbm, out_hbm, idx_v, raw_v, tr_v,
          sg0, sg1, so0, so1, si0, si1):
        sg = (sg0, sg1)
        so = (so0, so1)
        si = (si0, si1)
        wid = lax.axis_index("s") * NC + lax.axis_index("c")
        s0 = wid * per_w

        def start_idx(s, b):
            pltpu.async_copy(idx_hbm.at[pl.ds(s0 + s, 1)], idx_v.at[b], si[b])

        def wait_idx(b):
            pltpu.make_async_copy(idx_hbm.at[pl.ds(0, 1)], idx_v.at[b], si[b]).wait()

        def start_gather(b):
            pltpu.async_copy(table_hbm.at[idx_v.at[b, 0]], raw_v.at[b], sg[b])

        def wait_gather(b):
            pltpu.make_async_copy(table_hbm.at[pl.ds(0, _LB)], raw_v.at[b], sg[b]).wait()

        def start_out(s, b):
            sa = s0 + s
            h = sa // tb_per_h
            tb = sa % tb_per_h
            pltpu.async_copy(tr_v.at[b], out_hbm.at[h, :, pl.ds(tb * _LB, _LB)], so[b])

        def wait_out(b):
            pltpu.make_async_copy(
                tr_v.at[b], out_hbm.at[0, :, pl.ds(0, _LB)], so[b]
            ).wait()

        def transpose(b):
            def dbody(i, c):
                for dd in range(4):
                    d = i * 4 + dd
                    col = jnp.full((_LB // 8,), d, dtype=jnp.int32)
                    for bg in range(8):
                        rows = lax.iota(jnp.int32, 16) + bg * 16
                        v = plsc.load_gather(raw_v.at[b], [rows, col])
                        tr_v[b, d, pl.ds(bg * 16, 16)] = v
                return c

            lax.fori_loop(0, D // 4, dbody, 0)

        def pair(t, prefetch, first):
            g0 = 2 * t
            wait_gather(0)
            wait_idx(1)
            start_gather(1)
            if prefetch:
                start_idx(g0 + 2, 0)
            if not first:
                wait_out(0)
            transpose(0)
            start_out(g0, 0)
            wait_gather(1)
            if prefetch:
                wait_idx(0)
                start_gather(0)
                start_idx(g0 + 3, 1)
            if not first:
                wait_out(1)
            transpose(1)
            start_out(g0 + 1, 1)

        start_idx(0, 0)
        wait_idx(0)
        start_gather(0)
        start_idx(1, 1)
        T = per_w // 2
        pair(0, True, True)
        lax.fori_loop(1, T - 1, lambda t, c: (pair(t, True, False), c)[1], 0)
        pair(T - 1, False, False)
        wait_out(0)
        wait_out(1)

    return k


def kernel(input, table):
    B, H = input.shape
    V, D = table.shape
    idx2d = input.T.reshape(H * B // _LB, _LB).astype(jnp.int32)
    table_p = jnp.pad(table, ((0, 0), (0, _DP - D)))
    out = _make_gather(B, H, V, D)(idx2d, table_p)
    return jnp.transpose(out, (2, 0, 1))


# speedup vs baseline: 1.8877x; 1.8877x over previous
import functools

import jax
import jax.numpy as jnp
from jax import lax
from jax.experimental import pallas as pl
from jax.experimental.pallas import tpu as pltpu
from jax.experimental.pallas import tpu_sc as plsc

_LB = 128
_DP = 128


@functools.lru_cache(maxsize=None)
def _make_gather(B, H, V, D):
    n_slab = B * H // _LB
    info = plsc.get_sparse_core_info()
    NC, NS = info.num_cores, info.num_subcores
    NW = NC * NS
    per_w = n_slab // NW
    assert per_w * NW == n_slab and per_w % 2 == 0
    tb_per_h = B // _LB
    mesh = plsc.VectorSubcoreMesh(core_axis_name="c", subcore_axis_name="s")

    @functools.partial(
        pl.kernel,
        mesh=mesh,
        compiler_params=pltpu.CompilerParams(needs_layout_passes=False),
        out_type=jax.ShapeDtypeStruct((H, D, B), jnp.float32),
        scratch_types=[
            pltpu.VMEM((2, 1, _LB), jnp.int32),
            pltpu.VMEM((2, _LB, _DP), jnp.float32),
            pltpu.VMEM((2, D, _LB), jnp.float32),
            pltpu.SemaphoreType.DMA,
            pltpu.SemaphoreType.DMA,
            pltpu.SemaphoreType.DMA,
            pltpu.SemaphoreType.DMA,
            pltpu.SemaphoreType.DMA,
            pltpu.SemaphoreType.DMA,
        ],
    )
    def k(idx_hbm, table_hbm, out_hbm, idx_v, raw_v, tr_v,
          sg0, sg1, so0, so1, si0, si1):
        sg = (sg0, sg1)
        so = (so0, so1)
        si = (si0, si1)
        wid = lax.axis_index("s") * NC + lax.axis_index("c")
        s0 = wid * per_w

        def start_idx(s, b):
            pltpu.async_copy(idx_hbm.at[pl.ds(s0 + s, 1)], idx_v.at[b], si[b])

        def wait_idx(b):
            pltpu.make_async_copy(idx_hbm.at[pl.ds(0, 1)], idx_v.at[b], si[b]).wait()

        def start_gather(b):
            pltpu.async_copy(table_hbm.at[idx_v.at[b, 0]], raw_v.at[b], sg[b])

        def wait_gather(b):
            pltpu.make_async_copy(table_hbm.at[pl.ds(0, _LB)], raw_v.at[b], sg[b]).wait()

        def start_out(s, b):
            sa = s0 + s
            h = sa // tb_per_h
            tb = sa % tb_per_h
            pltpu.async_copy(tr_v.at[b], out_hbm.at[h, :, pl.ds(tb * _LB, _LB)], so[b])

        def wait_out(b):
            pltpu.make_async_copy(
                tr_v.at[b], out_hbm.at[0, :, pl.ds(0, _LB)], so[b]
            ).wait()

        def transpose(b):
            @plsc.parallel_loop(0, D, 1, unroll=8)
            def dbody(d):
                col = jnp.full((16,), d, dtype=jnp.int32)
                for bg in range(8):
                    rows = lax.iota(jnp.int32, 16) + bg * 16
                    v = plsc.load_gather(raw_v.at[b], [rows, col])
                    tr_v[b, d, pl.ds(bg * 16, 16)] = v

        def pair(t, prefetch, first):
            g0 = 2 * t
            wait_gather(0)
            wait_idx(1)
            start_gather(1)
            if prefetch:
                start_idx(g0 + 2, 0)
            if not first:
                wait_out(0)
            transpose(0)
            start_out(g0, 0)
            wait_gather(1)
            if prefetch:
                wait_idx(0)
                start_gather(0)
                start_idx(g0 + 3, 1)
            if not first:
                wait_out(1)
            transpose(1)
            start_out(g0 + 1, 1)

        start_idx(0, 0)
        wait_idx(0)
        start_gather(0)
        start_idx(1, 1)
        T = per_w // 2
        pair(0, True, True)
        lax.fori_loop(1, T - 1, lambda t, c: (pair(t, True, False), c)[1], 0)
        pair(T - 1, False, False)
        wait_out(0)
        wait_out(1)

    return k


def kernel(input, table):
    B, H = input.shape
    V, D = table.shape
    idx2d = input.T.reshape(H * B // _LB, _LB).astype(jnp.int32)
    table_p = jnp.pad(table, ((0, 0), (0, _DP - D)))
    out = _make_gather(B, H, V, D)(idx2d, table_p)
    return jnp.transpose(out, (2, 0, 1))
